# baseline (speedup 1.0000x reference)
import functools

import jax
import jax.numpy as jnp
from jax import lax
from jax.experimental import pallas as pl
from jax.experimental.pallas import tpu as pltpu
from jax.experimental.pallas import tpu_sc as plsc

B = 16384
NUM = 13
NCAT = 26
VOCAB = 100000
D = 32
NT = 1 + NUM + NCAT
NW = 32
NB = 32
ROWS_PER_W = B // NW
NCHUNKS = ROWS_PER_W // NB


def _tok_body(xnum_hbm, idx_hbm, cdst_hbm, hdst_hbm, w_hbm, b_hbm, cls_hbm,
              table_hbm, out_hbm,
              idx_v, cdst_v, hdst_v, xnum_v, cat_v, head_v, w_v, b_v, cls_v,
              gsem, ssem):
    wid = lax.axis_index("s") * 2 + lax.axis_index("c")
    base = wid * ROWS_PER_W

    pltpu.sync_copy(w_hbm, w_v)
    pltpu.sync_copy(b_hbm, b_v)
    pltpu.sync_copy(cls_hbm, cls_v)
    cls0 = cls_v[pl.ds(0, 16)]
    cls1 = cls_v[pl.ds(16, 16)]

    def chunk_body(c, carry):
        row0 = base + c * NB
        pltpu.sync_copy(idx_hbm.at[pl.ds(row0 * NCAT, NB * NCAT)], idx_v)
        pltpu.sync_copy(cdst_hbm.at[pl.ds(row0 * NCAT, NB * NCAT)], cdst_v)
        pltpu.sync_copy(hdst_hbm.at[pl.ds(row0 * (1 + NUM), NB * (1 + NUM))],
                        hdst_v)
        pltpu.sync_copy(xnum_hbm.at[pl.ds(row0 * NUM, NB * NUM)], xnum_v)
        gcopy = pltpu.async_copy(table_hbm.at[idx_v], cat_v, gsem)

        def row_body(i, carry2):
            head_v[i * (1 + NUM), pl.ds(0, 16)] = cls0
            head_v[i * (1 + NUM), pl.ds(16, 16)] = cls1
            for j in range(NUM):
                xij = plsc.load_gather(
                    xnum_v, [jnp.full((16,), i * NUM + j, jnp.int32)])
                for h in range(2):
                    off = (2 * j + h) * 16
                    head_v[i * (1 + NUM) + 1 + j, pl.ds(h * 16, 16)] = (
                        xij * w_v[pl.ds(off, 16)] + b_v[pl.ds(off, 16)])
            return carry2

        lax.fori_loop(0, NB, row_body, 0)
        gcopy.wait()
        s1 = pltpu.async_copy(cat_v, out_hbm.at[cdst_v], ssem)
        s2 = pltpu.async_copy(head_v, out_hbm.at[hdst_v], ssem)
        s1.wait()
        s2.wait()
        return carry

    lax.fori_loop(0, NCHUNKS, chunk_body, 0)


@functools.partial(
    pl.kernel,
    out_type=jax.ShapeDtypeStruct((B * NT, D), jnp.float32),
    mesh=plsc.VectorSubcoreMesh(core_axis_name="c", subcore_axis_name="s"),
    compiler_params=pltpu.CompilerParams(
        needs_layout_passes=False, use_tc_tiling_on_sc=False),
    scratch_types=[
        pltpu.VMEM((NB * NCAT,), jnp.int32),
        pltpu.VMEM((NB * NCAT,), jnp.int32),
        pltpu.VMEM((NB * (1 + NUM),), jnp.int32),
        pltpu.VMEM((NB * NUM,), jnp.float32),
        pltpu.VMEM((NB * NCAT, D), jnp.float32),
        pltpu.VMEM((NB * (1 + NUM), D), jnp.float32),
        pltpu.VMEM((NUM * D,), jnp.float32),
        pltpu.VMEM((NUM * D,), jnp.float32),
        pltpu.VMEM((D,), jnp.float32),
        pltpu.SemaphoreType.DMA,
        pltpu.SemaphoreType.DMA,
    ],
)
def _tok_kernel(*refs):
    _tok_body(*refs)


def kernel(x_num, x_cat, num_weights, num_bias, cat_tables, cls_token):
    idx = (x_cat + (jnp.arange(NCAT, dtype=jnp.int32) * VOCAB)[None, :])
    brow = jnp.arange(B, dtype=jnp.int32)[:, None] * NT
    cdst = (brow + (1 + NUM) + jnp.arange(NCAT, dtype=jnp.int32)[None, :])
    hdst = (brow + jnp.arange(1 + NUM, dtype=jnp.int32)[None, :])
    out = _tok_kernel(
        x_num.reshape(-1),
        idx.reshape(-1),
        cdst.reshape(-1),
        hdst.reshape(-1),
        num_weights.reshape(-1),
        num_bias.reshape(-1),
        cls_token.reshape(-1),
        cat_tables.reshape(NCAT * VOCAB, D),
    )
    return out.reshape(B, NT, D)
